# baseline (speedup 1.0000x reference)
import functools

import jax
import jax.numpy as jnp
from jax import lax
from jax.experimental import pallas as pl
from jax.experimental.pallas import tpu as pltpu
from jax.experimental.pallas import tpu_sc as plsc

MAXF = 20
TK = 3
NS = 500
SIG = 0.05
NPAD = 256
NSC = 208

_CONSTS = {}


def _noise_sc(rows, n):
    ck = ("sc", rows, n)
    if ck not in _CONSTS:
        nz = jax.random.normal(jax.random.key(1), (rows, NS, n), dtype=jnp.float32)
        nz = nz * SIG
        pad = jnp.full((rows, NS, NSC - n), -1e30, dtype=jnp.float32)
        _CONSTS[ck] = jnp.concatenate([nz, pad], axis=-1)
    return _CONSTS[ck]


def _ln(v, g, b):
    m = v.mean(axis=-1, keepdims=True)
    var = ((v - m) * (v - m)).mean(axis=-1, keepdims=True)
    return (v - m) / jnp.sqrt(var + 1e-5) * g + b


def _gelu(v):
    return 0.5 * v * (1.0 + jax.lax.erf(v * 0.7071067811865476))


def _score_body(xf_ref, gf_ref, gs_ref, ln1g_ref, ln1b_ref, w1_ref, ln2g_ref,
                ln2b_ref, w2_ref, wo1_ref, wo2_ref, score_ref):
    rows, n, _ = xf_ref.shape
    dh = w1_ref.shape[1]
    xf = xf_ref[...].reshape(rows * n, xf_ref.shape[2])
    xi = _gelu(jnp.dot(_ln(xf, ln1g_ref[...], ln1b_ref[...]), w1_ref[...],
                       preferred_element_type=jnp.float32))

    def head(g_ref):
        gi = _gelu(jnp.dot(_ln(g_ref[...], ln2g_ref[...], ln2b_ref[...]),
                           w2_ref[...], preferred_element_type=jnp.float32))
        gi_b = jnp.broadcast_to(gi[:, None, :], (rows, n, dh)).reshape(rows * n, dh)
        h = jnp.concatenate([xi, gi_b], axis=-1)
        o = _gelu(jnp.dot(h, wo1_ref[...], preferred_element_type=jnp.float32))
        s = jnp.tanh(jnp.dot(o, wo2_ref[...], preferred_element_type=jnp.float32))
        return s.reshape(rows, n)

    sc = head(gf_ref) + head(gs_ref)
    score_ref[...] = jnp.concatenate(
        [sc, jnp.zeros((rows, NPAD - n), dtype=jnp.float32)], axis=-1)[:, None, :]


def _sc_topk_body(score_hbm, noise_hbm, cnt_hbm, score_v, nz_v, cnt_v):
    wid = lax.axis_index("s") * 2 + lax.axis_index("c")
    rows_per_w = 5
    lane = lax.iota(jnp.int32, 16)
    neg = jnp.float32(-3.0e38)
    zeros16 = jnp.zeros((16,), jnp.float32)
    ones16 = jnp.ones((16,), jnp.float32)

    def row_body(r, _carry):
        row = wid * rows_per_w + r
        pltpu.sync_copy(score_hbm.at[row], score_v)
        pltpu.sync_copy(noise_hbm.at[row], nz_v)
        for k in range(3 * NSC // 16):
            cnt_v[pl.ds(16 * k, 16)] = zeros16
        s_vecs = [score_v[pl.ds(16 * j, 16)] for j in range(NSC // 16)]

        def samp_body(s, _c2):
            t1 = jnp.full((16,), neg)
            t2 = jnp.full((16,), neg)
            t3 = jnp.full((16,), neg)
            i1 = jnp.zeros((16,), jnp.int32)
            i2 = jnp.zeros((16,), jnp.int32)
            i3 = jnp.zeros((16,), jnp.int32)
            for j in range(NSC // 16):
                v = nz_v[s, pl.ds(16 * j, 16)] + s_vecs[j]
                idx = lane + (16 * j)
                c1 = v > t1
                c2 = v > t2
                c3 = v > t3
                t3n = jnp.where(c2, t2, jnp.where(c3, v, t3))
                i3n = jnp.where(c2, i2, jnp.where(c3, idx, i3))
                t2n = jnp.where(c1, t1, jnp.where(c2, v, t2))
                i2n = jnp.where(c1, i1, jnp.where(c2, idx, i2))
                t1 = jnp.where(c1, v, t1)
                i1 = jnp.where(c1, idx, i1)
                t2, t3, i2, i3 = t2n, t3n, i2n, i3n
            picks = []
            for _k in range(TK):
                m = jnp.max(t1)
                ik = jnp.min(jnp.where(t1 == m, i1, 100000))
                picks.append(ik)
                hit = (t1 == m) & (i1 == ik)
                t1 = jnp.where(hit, t2, t1)
                i1 = jnp.where(hit, i2, i1)
                t2 = jnp.where(hit, t3, t2)
                i2 = jnp.where(hit, i3, i2)
                t3 = jnp.where(hit, neg, t3)
            a, b, c = picks
            lo = jnp.minimum(jnp.minimum(a, b), c)
            hi = jnp.maximum(jnp.maximum(a, b), c)
            mid = a + b + c - lo - hi
            idxv = jnp.where(lane == 0, lo,
                             jnp.where(lane == 1, mid + NSC, hi + 2 * NSC))
            plsc.addupdate_scatter(cnt_v, [idxv], ones16, mask=lane < TK)
            return _c2

        lax.fori_loop(0, NS, samp_body, 0)
        pltpu.sync_copy(cnt_v, cnt_hbm.at[row])
        return _carry

    lax.fori_loop(0, rows_per_w, row_body, 0)


def _select_body(cnt_ref, xr_ref, sel_ref):
    n = xr_ref.shape[0]
    ind = cnt_ref[...] * (1.0 / NS)
    sel_ref[...] = jnp.dot(ind[:, :n], xr_ref[...],
                           preferred_element_type=jnp.float32)


def kernel(x, guidance_frame, guidance_sentence, ln1_g, ln1_b, W1, ln2_g,
           ln2_b, W2, Wo1, Wo2):
    B, L, D = x.shape
    n = L // MAXF
    rows = B * MAXF
    dh = W1.shape[1]

    xr = x.reshape(rows, n, D)
    gf = guidance_frame.reshape(rows, D)
    gs = jnp.broadcast_to(guidance_sentence, (B, MAXF, D)).reshape(rows, D)

    CH = 8
    gsteps = rows // CH

    def _w(shape):
        nd = len(shape)
        return pl.BlockSpec(shape, lambda i, _nd=nd: (0,) * _nd)

    score = pl.pallas_call(
        _score_body,
        grid=(gsteps,),
        in_specs=[
            pl.BlockSpec((CH, n, D), lambda i: (i, 0, 0)),
            pl.BlockSpec((CH, D), lambda i: (i, 0)),
            pl.BlockSpec((CH, D), lambda i: (i, 0)),
            _w((D,)), _w((D,)), _w((D, dh)),
            _w((D,)), _w((D,)), _w((D, dh)),
            _w((2 * dh, dh)), _w((dh, 1)),
        ],
        out_specs=pl.BlockSpec((CH, 1, NPAD), lambda i: (i, 0, 0)),
        out_shape=jax.ShapeDtypeStruct((rows, 1, NPAD), jnp.float32),
    )(xr, gf, gs, ln1_g, ln1_b, W1, ln2_g, ln2_b, W2, Wo1, Wo2)

    score2d = score.reshape(rows, NPAD)
    nzp = _noise_sc(rows, n)

    mesh = plsc.VectorSubcoreMesh(core_axis_name="c", subcore_axis_name="s")
    sc_topk = functools.partial(
        pl.kernel,
        out_type=jax.ShapeDtypeStruct((rows, TK * NSC), jnp.float32),
        mesh=mesh,
        compiler_params=pltpu.CompilerParams(needs_layout_passes=False),
        scratch_types=[
            pltpu.VMEM((NPAD,), jnp.float32),
            pltpu.VMEM((NS, NSC), jnp.float32),
            pltpu.VMEM((TK * NSC,), jnp.float32),
        ],
    )(_sc_topk_body)
    cnt = sc_topk(score2d, nzp)

    cnt3 = cnt.reshape(rows, TK, NSC)
    sel = pl.pallas_call(
        _select_body,
        grid=(rows,),
        in_specs=[
            pl.BlockSpec((None, TK, NSC), lambda i: (i, 0, 0)),
            pl.BlockSpec((None, n, D), lambda i: (i, 0, 0)),
        ],
        out_specs=pl.BlockSpec((None, TK, D), lambda i: (i, 0, 0)),
        out_shape=jax.ShapeDtypeStruct((rows, TK, D), jnp.float32),
    )(cnt3, xr)

    return sel.reshape(B, MAXF, TK, D)


# speedup vs baseline: 1.0527x; 1.0527x over previous
import functools

import jax
import jax.numpy as jnp
from jax import lax
from jax.experimental import pallas as pl
from jax.experimental.pallas import tpu as pltpu
from jax.experimental.pallas import tpu_sc as plsc

MAXF = 20
TK = 3
NS = 500
SIG = 0.05
NPAD = 256
NSC = 208

_CONSTS = {}


def _noise_sc(rows, n):
    ck = ("sc", rows, n)
    if ck not in _CONSTS:
        nz = jax.random.normal(jax.random.key(1), (rows, NS, n), dtype=jnp.float32)
        nz = nz * SIG
        pad = jnp.full((rows, NS, NSC - n), -1e30, dtype=jnp.float32)
        _CONSTS[ck] = jnp.concatenate([nz, pad], axis=-1)
    return _CONSTS[ck]


def _ln(v, g, b):
    m = v.mean(axis=-1, keepdims=True)
    var = ((v - m) * (v - m)).mean(axis=-1, keepdims=True)
    return (v - m) / jnp.sqrt(var + 1e-5) * g + b


def _gelu(v):
    return 0.5 * v * (1.0 + jax.lax.erf(v * 0.7071067811865476))


def _score_body(xf_ref, gf_ref, gs_ref, ln1g_ref, ln1b_ref, w1_ref, ln2g_ref,
                ln2b_ref, w2_ref, wo1_ref, wo2_ref, score_ref):
    rows, n, _ = xf_ref.shape
    dh = w1_ref.shape[1]
    xf = xf_ref[...].reshape(rows * n, xf_ref.shape[2])
    xi = _gelu(jnp.dot(_ln(xf, ln1g_ref[...], ln1b_ref[...]), w1_ref[...],
                       preferred_element_type=jnp.float32))

    def head(g_ref):
        gi = _gelu(jnp.dot(_ln(g_ref[...], ln2g_ref[...], ln2b_ref[...]),
                           w2_ref[...], preferred_element_type=jnp.float32))
        gi_b = jnp.broadcast_to(gi[:, None, :], (rows, n, dh)).reshape(rows * n, dh)
        h = jnp.concatenate([xi, gi_b], axis=-1)
        o = _gelu(jnp.dot(h, wo1_ref[...], preferred_element_type=jnp.float32))
        s = jnp.tanh(jnp.dot(o, wo2_ref[...], preferred_element_type=jnp.float32))
        return s.reshape(rows, n)

    sc = head(gf_ref) + head(gs_ref)
    score_ref[...] = jnp.concatenate(
        [sc, jnp.zeros((rows, NPAD - n), dtype=jnp.float32)], axis=-1)[:, None, :]


def _sc_topk_body(score_hbm, noise_hbm, cnt_hbm, score_v, nz_v, cnt_v):
    wid = lax.axis_index("s") * 2 + lax.axis_index("c")
    rows_per_w = 5
    lane = lax.iota(jnp.int32, 16)
    neg = jnp.float32(-3.0e38)
    zeros16 = jnp.zeros((16,), jnp.float32)
    ones16 = jnp.ones((16,), jnp.float32)

    def row_body(r, _carry):
        row = wid * rows_per_w + r
        pltpu.sync_copy(score_hbm.at[row], score_v)
        pltpu.sync_copy(noise_hbm.at[row], nz_v)
        for k in range(3 * NSC // 16):
            cnt_v[pl.ds(16 * k, 16)] = zeros16
        s_vecs = [score_v[pl.ds(16 * j, 16)] for j in range(NSC // 16)]

        def one_sample(s):
            t1 = jnp.full((16,), neg)
            t2 = jnp.full((16,), neg)
            t3 = jnp.full((16,), neg)
            i1 = jnp.zeros((16,), jnp.int32)
            i2 = jnp.zeros((16,), jnp.int32)
            i3 = jnp.zeros((16,), jnp.int32)
            for j in range(NSC // 16):
                v = nz_v[s, pl.ds(16 * j, 16)] + s_vecs[j]
                idx = lane + (16 * j)
                c1 = v > t1
                c2 = v > t2
                c3 = v > t3
                t3n = jnp.where(c2, t2, jnp.where(c3, v, t3))
                i3n = jnp.where(c2, i2, jnp.where(c3, idx, i3))
                t2n = jnp.where(c1, t1, jnp.where(c2, v, t2))
                i2n = jnp.where(c1, i1, jnp.where(c2, idx, i2))
                t1 = jnp.where(c1, v, t1)
                i1 = jnp.where(c1, idx, i1)
                t2, t3, i2, i3 = t2n, t3n, i2n, i3n
            picks = []
            for _k in range(TK):
                m = jnp.max(t1)
                mask = t1 == m
                lane0 = plsc.all_reduce_ffs(mask)
                ikv = jnp.min(jnp.where(mask, i1, 100000))
                picks.append(ikv)
                hit = mask & (lane == lane0)
                t1 = jnp.where(hit, t2, t1)
                i1 = jnp.where(hit, i2, i1)
                t2 = jnp.where(hit, t3, t2)
                i2 = jnp.where(hit, i3, i2)
                t3 = jnp.where(hit, neg, t3)
            a, b, c = picks
            lo = jnp.minimum(jnp.minimum(a, b), c)
            hi = jnp.maximum(jnp.maximum(a, b), c)
            mid = a + b + c - lo - hi
            return jnp.where(lane == 0, lo,
                             jnp.where(lane == 1, mid + NSC, hi + 2 * NSC))

        def samp_body(s, _c2):
            for u in range(4):
                iv = one_sample(4 * s + u)
                plsc.addupdate_scatter(cnt_v, [iv], ones16, mask=lane < TK)
            return _c2

        lax.fori_loop(0, NS // 4, samp_body, 0)
        pltpu.sync_copy(cnt_v, cnt_hbm.at[row])
        return _carry

    lax.fori_loop(0, rows_per_w, row_body, 0)


def _select_body(cnt_ref, xr_ref, sel_ref):
    n = xr_ref.shape[0]
    ind = cnt_ref[...] * (1.0 / NS)
    sel_ref[...] = jnp.dot(ind[:, :n], xr_ref[...],
                           preferred_element_type=jnp.float32)


def kernel(x, guidance_frame, guidance_sentence, ln1_g, ln1_b, W1, ln2_g,
           ln2_b, W2, Wo1, Wo2):
    B, L, D = x.shape
    n = L // MAXF
    rows = B * MAXF
    dh = W1.shape[1]

    xr = x.reshape(rows, n, D)
    gf = guidance_frame.reshape(rows, D)
    gs = jnp.broadcast_to(guidance_sentence, (B, MAXF, D)).reshape(rows, D)

    CH = 8
    gsteps = rows // CH

    def _w(shape):
        nd = len(shape)
        return pl.BlockSpec(shape, lambda i, _nd=nd: (0,) * _nd)

    score = pl.pallas_call(
        _score_body,
        grid=(gsteps,),
        in_specs=[
            pl.BlockSpec((CH, n, D), lambda i: (i, 0, 0)),
            pl.BlockSpec((CH, D), lambda i: (i, 0)),
            pl.BlockSpec((CH, D), lambda i: (i, 0)),
            _w((D,)), _w((D,)), _w((D, dh)),
            _w((D,)), _w((D,)), _w((D, dh)),
            _w((2 * dh, dh)), _w((dh, 1)),
        ],
        out_specs=pl.BlockSpec((CH, 1, NPAD), lambda i: (i, 0, 0)),
        out_shape=jax.ShapeDtypeStruct((rows, 1, NPAD), jnp.float32),
    )(xr, gf, gs, ln1_g, ln1_b, W1, ln2_g, ln2_b, W2, Wo1, Wo2)

    score2d = score.reshape(rows, NPAD)
    nzp = _noise_sc(rows, n)

    mesh = plsc.VectorSubcoreMesh(core_axis_name="c", subcore_axis_name="s")
    sc_topk = functools.partial(
        pl.kernel,
        out_type=jax.ShapeDtypeStruct((rows, TK * NSC), jnp.float32),
        mesh=mesh,
        compiler_params=pltpu.CompilerParams(needs_layout_passes=False),
        scratch_types=[
            pltpu.VMEM((NPAD,), jnp.float32),
            pltpu.VMEM((NS, NSC), jnp.float32),
            pltpu.VMEM((TK * NSC,), jnp.float32),
        ],
    )(_sc_topk_body)
    cnt = sc_topk(score2d, nzp)

    cnt3 = cnt.reshape(rows, TK, NSC)
    sel = pl.pallas_call(
        _select_body,
        grid=(rows,),
        in_specs=[
            pl.BlockSpec((None, TK, NSC), lambda i: (i, 0, 0)),
            pl.BlockSpec((None, n, D), lambda i: (i, 0, 0)),
        ],
        out_specs=pl.BlockSpec((None, TK, D), lambda i: (i, 0, 0)),
        out_shape=jax.ShapeDtypeStruct((rows, TK, D), jnp.float32),
    )(cnt3, xr)

    return sel.reshape(B, MAXF, TK, D)


# speedup vs baseline: 1.1420x; 1.0849x over previous
import functools

import jax
import jax.numpy as jnp
from jax import lax
from jax.experimental import pallas as pl
from jax.experimental.pallas import tpu as pltpu
from jax.experimental.pallas import tpu_sc as plsc

MAXF = 20
TK = 3
NS = 500
SIG = 0.05
NPAD = 256
NSC = 208

_CONSTS = {}


def _noise_sc(rows, n):
    ck = ("sc", rows, n)
    if ck not in _CONSTS:
        nz = jax.random.normal(jax.random.key(1), (rows, NS, n), dtype=jnp.float32)
        nz = nz * SIG
        pad = jnp.full((rows, NS, NSC - n), -1e30, dtype=jnp.float32)
        _CONSTS[ck] = jnp.concatenate([nz, pad], axis=-1)
    return _CONSTS[ck]


def _ln(v, g, b):
    m = v.mean(axis=-1, keepdims=True)
    var = ((v - m) * (v - m)).mean(axis=-1, keepdims=True)
    return (v - m) / jnp.sqrt(var + 1e-5) * g + b


def _gelu(v):
    return 0.5 * v * (1.0 + jax.lax.erf(v * 0.7071067811865476))


def _score_body(xf_ref, gf_ref, gs_ref, ln1g_ref, ln1b_ref, w1_ref, ln2g_ref,
                ln2b_ref, w2_ref, wo1_ref, wo2_ref, score_ref):
    rows, n, _ = xf_ref.shape
    dh = w1_ref.shape[1]
    xf = xf_ref[...].reshape(rows * n, xf_ref.shape[2])
    xi = _gelu(jnp.dot(_ln(xf, ln1g_ref[...], ln1b_ref[...]), w1_ref[...],
                       preferred_element_type=jnp.float32))

    def head(g_ref):
        gi = _gelu(jnp.dot(_ln(g_ref[...], ln2g_ref[...], ln2b_ref[...]),
                           w2_ref[...], preferred_element_type=jnp.float32))
        gi_b = jnp.broadcast_to(gi[:, None, :], (rows, n, dh)).reshape(rows * n, dh)
        h = jnp.concatenate([xi, gi_b], axis=-1)
        o = _gelu(jnp.dot(h, wo1_ref[...], preferred_element_type=jnp.float32))
        s = jnp.tanh(jnp.dot(o, wo2_ref[...], preferred_element_type=jnp.float32))
        return s.reshape(rows, n)

    sc = head(gf_ref) + head(gs_ref)
    score_ref[...] = jnp.concatenate(
        [sc, jnp.zeros((rows, NPAD - n), dtype=jnp.float32)], axis=-1)[:, None, :]


def _sc_topk_body(score_hbm, noise_hbm, cnt_hbm, score_v, nz_v, cnt_v):
    wid = lax.axis_index("s") * 2 + lax.axis_index("c")
    rows_per_w = 5
    lane = lax.iota(jnp.int32, 16)
    neg = jnp.float32(-3.0e38)
    zeros16 = jnp.zeros((16,), jnp.float32)
    ones16 = jnp.ones((16,), jnp.float32)

    def row_body(r, _carry):
        row = wid * rows_per_w + r
        pltpu.sync_copy(score_hbm.at[row], score_v)
        pltpu.sync_copy(noise_hbm.at[row], nz_v)
        for k in range(3 * NSC // 16):
            cnt_v[pl.ds(16 * k, 16)] = zeros16
        s_vecs = [score_v[pl.ds(16 * j, 16)] for j in range(NSC // 16)]

        def one_sample(s):
            t1 = jnp.full((16,), neg)
            t2 = jnp.full((16,), neg)
            t3 = jnp.full((16,), neg)
            i1 = jnp.zeros((16,), jnp.int32)
            i2 = jnp.zeros((16,), jnp.int32)
            i3 = jnp.zeros((16,), jnp.int32)
            for j in range(NSC // 16):
                v = nz_v[s, pl.ds(16 * j, 16)] + s_vecs[j]
                idx = lane + (16 * j)
                c1 = v > t1
                c2 = v > t2
                c3 = v > t3
                t3n = jnp.where(c2, t2, jnp.where(c3, v, t3))
                i3n = jnp.where(c2, i2, jnp.where(c3, idx, i3))
                t2n = jnp.where(c1, t1, jnp.where(c2, v, t2))
                i2n = jnp.where(c1, i1, jnp.where(c2, idx, i2))
                t1 = jnp.where(c1, v, t1)
                i1 = jnp.where(c1, idx, i1)
                t2, t3, i2, i3 = t2n, t3n, i2n, i3n
            picks = []
            for _k in range(TK):
                m = jnp.max(t1)
                mask = t1 == m
                lane0 = plsc.all_reduce_ffs(mask)
                ikv = jnp.min(jnp.where(mask, i1, 100000))
                picks.append(ikv)
                hit = mask & (lane == lane0)
                t1 = jnp.where(hit, t2, t1)
                i1 = jnp.where(hit, i2, i1)
                t2 = jnp.where(hit, t3, t2)
                i2 = jnp.where(hit, i3, i2)
                t3 = jnp.where(hit, neg, t3)
            a, b, c = picks
            lo = jnp.minimum(jnp.minimum(a, b), c)
            hi = jnp.maximum(jnp.maximum(a, b), c)
            mid = a + b + c - lo - hi
            return jnp.where(lane == 0, lo,
                             jnp.where(lane == 1, mid + NSC, hi + 2 * NSC))

        def samp_body(s, _c2):
            for u in range(4):
                iv = one_sample(4 * s + u)
                plsc.addupdate_scatter(cnt_v, [iv], ones16, mask=lane < TK)
            return _c2

        lax.fori_loop(0, NS // 4, samp_body, 0)
        pltpu.sync_copy(cnt_v, cnt_hbm.at[row])
        return _carry

    lax.fori_loop(0, rows_per_w, row_body, 0)


def _select_body(cnt_ref, xr_ref, sel_ref):
    ch, n, _ = xr_ref.shape
    ind = cnt_ref[...] * (1.0 / NS)
    for r in range(ch):
        sel_ref[r] = jnp.dot(ind[r, :, :n], xr_ref[r],
                             preferred_element_type=jnp.float32)


def kernel(x, guidance_frame, guidance_sentence, ln1_g, ln1_b, W1, ln2_g,
           ln2_b, W2, Wo1, Wo2):
    B, L, D = x.shape
    n = L // MAXF
    rows = B * MAXF
    dh = W1.shape[1]

    xr = x.reshape(rows, n, D)
    gf = guidance_frame.reshape(rows, D)
    gs = jnp.broadcast_to(guidance_sentence, (B, MAXF, D)).reshape(rows, D)

    CH = 8
    gsteps = rows // CH

    def _w(shape):
        nd = len(shape)
        return pl.BlockSpec(shape, lambda i, _nd=nd: (0,) * _nd)

    score = pl.pallas_call(
        _score_body,
        grid=(gsteps,),
        in_specs=[
            pl.BlockSpec((CH, n, D), lambda i: (i, 0, 0)),
            pl.BlockSpec((CH, D), lambda i: (i, 0)),
            pl.BlockSpec((CH, D), lambda i: (i, 0)),
            _w((D,)), _w((D,)), _w((D, dh)),
            _w((D,)), _w((D,)), _w((D, dh)),
            _w((2 * dh, dh)), _w((dh, 1)),
        ],
        out_specs=pl.BlockSpec((CH, 1, NPAD), lambda i: (i, 0, 0)),
        out_shape=jax.ShapeDtypeStruct((rows, 1, NPAD), jnp.float32),
    )(xr, gf, gs, ln1_g, ln1_b, W1, ln2_g, ln2_b, W2, Wo1, Wo2)

    score2d = score.reshape(rows, NPAD)
    nzp = _noise_sc(rows, n)

    mesh = plsc.VectorSubcoreMesh(core_axis_name="c", subcore_axis_name="s")
    sc_topk = functools.partial(
        pl.kernel,
        out_type=jax.ShapeDtypeStruct((rows, TK * NSC), jnp.float32),
        mesh=mesh,
        compiler_params=pltpu.CompilerParams(needs_layout_passes=False),
        scratch_types=[
            pltpu.VMEM((NPAD,), jnp.float32),
            pltpu.VMEM((NS, NSC), jnp.float32),
            pltpu.VMEM((TK * NSC,), jnp.float32),
        ],
    )(_sc_topk_body)
    cnt = sc_topk(score2d, nzp)

    cnt3 = cnt.reshape(rows, TK, NSC)
    sel = pl.pallas_call(
        _select_body,
        grid=(gsteps,),
        in_specs=[
            pl.BlockSpec((CH, TK, NSC), lambda i: (i, 0, 0)),
            pl.BlockSpec((CH, n, D), lambda i: (i, 0, 0)),
        ],
        out_specs=pl.BlockSpec((CH, TK, D), lambda i: (i, 0, 0)),
        out_shape=jax.ShapeDtypeStruct((rows, TK, D), jnp.float32),
    )(cnt3, xr)

    return sel.reshape(B, MAXF, TK, D)


# speedup vs baseline: 2.6169x; 2.2914x over previous
import functools

import jax
import jax.numpy as jnp
from jax import lax
from jax.experimental import pallas as pl
from jax.experimental.pallas import tpu as pltpu
from jax.experimental.pallas import tpu_sc as plsc

MAXF = 20
TK = 3
NS = 500
SIG = 0.05
NPAD = 256
NSC = 208

_CONSTS = {}


def _noise_sc(rows, n):
    ck = ("sc", rows, n)
    if ck not in _CONSTS:
        with jax.ensure_compile_time_eval():
            nz = jax.random.normal(jax.random.key(1), (rows, NS, n),
                                   dtype=jnp.float32)
            nz = nz * SIG
            pad = jnp.full((rows, NS, NSC - n), -1e30, dtype=jnp.float32)
            _CONSTS[ck] = jnp.concatenate([nz, pad], axis=-1)
    return _CONSTS[ck]


def _ln(v, g, b):
    m = v.mean(axis=-1, keepdims=True)
    var = ((v - m) * (v - m)).mean(axis=-1, keepdims=True)
    return (v - m) / jnp.sqrt(var + 1e-5) * g + b


def _gelu(v):
    return 0.5 * v * (1.0 + jax.lax.erf(v * 0.7071067811865476))


def _score_body(xf_ref, gf_ref, gs_ref, ln1g_ref, ln1b_ref, w1_ref, ln2g_ref,
                ln2b_ref, w2_ref, wo1_ref, wo2_ref, score_ref):
    rows, n, _ = xf_ref.shape
    dh = w1_ref.shape[1]
    xf = xf_ref[...].reshape(rows * n, xf_ref.shape[2])
    xi = _gelu(jnp.dot(_ln(xf, ln1g_ref[...], ln1b_ref[...]), w1_ref[...],
                       preferred_element_type=jnp.float32))

    def head(g_ref):
        gi = _gelu(jnp.dot(_ln(g_ref[...], ln2g_ref[...], ln2b_ref[...]),
                           w2_ref[...], preferred_element_type=jnp.float32))
        gi_b = jnp.broadcast_to(gi[:, None, :], (rows, n, dh)).reshape(rows * n, dh)
        h = jnp.concatenate([xi, gi_b], axis=-1)
        o = _gelu(jnp.dot(h, wo1_ref[...], preferred_element_type=jnp.float32))
        s = jnp.tanh(jnp.dot(o, wo2_ref[...], preferred_element_type=jnp.float32))
        return s.reshape(rows, n)

    sc = head(gf_ref) + head(gs_ref)
    score_ref[...] = jnp.concatenate(
        [sc, jnp.zeros((rows, NPAD - n), dtype=jnp.float32)], axis=-1)[:, None, :]


def _sc_topk_body(score_hbm, noise_hbm, cnt_hbm, score_v, nz_v, cnt_v):
    wid = lax.axis_index("s") * 2 + lax.axis_index("c")
    rows_per_w = 5
    lane = lax.iota(jnp.int32, 16)
    neg = jnp.float32(-3.0e38)
    zeros16 = jnp.zeros((16,), jnp.float32)
    ones16 = jnp.ones((16,), jnp.float32)

    def row_body(r, _carry):
        row = wid * rows_per_w + r
        pltpu.sync_copy(score_hbm.at[row], score_v)
        pltpu.sync_copy(noise_hbm.at[row], nz_v)
        for k in range(3 * NSC // 16):
            cnt_v[pl.ds(16 * k, 16)] = zeros16
        s_vecs = [score_v[pl.ds(16 * j, 16)] for j in range(NSC // 16)]

        def one_sample(s):
            t1 = jnp.full((16,), neg)
            t2 = jnp.full((16,), neg)
            t3 = jnp.full((16,), neg)
            i1 = jnp.zeros((16,), jnp.int32)
            i2 = jnp.zeros((16,), jnp.int32)
            i3 = jnp.zeros((16,), jnp.int32)
            for j in range(NSC // 16):
                v = nz_v[s, pl.ds(16 * j, 16)] + s_vecs[j]
                idx = lane + (16 * j)
                c1 = v > t1
                c2 = v > t2
                c3 = v > t3
                t3n = jnp.where(c2, t2, jnp.where(c3, v, t3))
                i3n = jnp.where(c2, i2, jnp.where(c3, idx, i3))
                t2n = jnp.where(c1, t1, jnp.where(c2, v, t2))
                i2n = jnp.where(c1, i1, jnp.where(c2, idx, i2))
                t1 = jnp.where(c1, v, t1)
                i1 = jnp.where(c1, idx, i1)
                t2, t3, i2, i3 = t2n, t3n, i2n, i3n
            picks = []
            for _k in range(TK):
                m = jnp.max(t1)
                mask = t1 == m
                lane0 = plsc.all_reduce_ffs(mask)
                ikv = jnp.min(jnp.where(mask, i1, 100000))
                picks.append(ikv)
                hit = mask & (lane == lane0)
                t1 = jnp.where(hit, t2, t1)
                i1 = jnp.where(hit, i2, i1)
                t2 = jnp.where(hit, t3, t2)
                i2 = jnp.where(hit, i3, i2)
                t3 = jnp.where(hit, neg, t3)
            a, b, c = picks
            lo = jnp.minimum(jnp.minimum(a, b), c)
            hi = jnp.maximum(jnp.maximum(a, b), c)
            mid = a + b + c - lo - hi
            return jnp.where(lane == 0, lo,
                             jnp.where(lane == 1, mid + NSC, hi + 2 * NSC))

        def samp_body(s, _c2):
            for u in range(4):
                iv = one_sample(4 * s + u)
                plsc.addupdate_scatter(cnt_v, [iv], ones16, mask=lane < TK)
            return _c2

        lax.fori_loop(0, NS // 4, samp_body, 0)
        pltpu.sync_copy(cnt_v, cnt_hbm.at[row])
        return _carry

    lax.fori_loop(0, rows_per_w, row_body, 0)


def _select_body(cnt_ref, xr_ref, sel_ref):
    ch, n, _ = xr_ref.shape
    ind = cnt_ref[...] * (1.0 / NS)
    for r in range(ch):
        sel_ref[r] = jnp.dot(ind[r, :, :n], xr_ref[r],
                             preferred_element_type=jnp.float32)


def kernel(x, guidance_frame, guidance_sentence, ln1_g, ln1_b, W1, ln2_g,
           ln2_b, W2, Wo1, Wo2):
    B, L, D = x.shape
    n = L // MAXF
    rows = B * MAXF
    dh = W1.shape[1]

    xr = x.reshape(rows, n, D)
    gf = guidance_frame.reshape(rows, D)
    gs = jnp.broadcast_to(guidance_sentence, (B, MAXF, D)).reshape(rows, D)

    CH = 8
    gsteps = rows // CH

    def _w(shape):
        nd = len(shape)
        return pl.BlockSpec(shape, lambda i, _nd=nd: (0,) * _nd)

    score = pl.pallas_call(
        _score_body,
        grid=(gsteps,),
        in_specs=[
            pl.BlockSpec((CH, n, D), lambda i: (i, 0, 0)),
            pl.BlockSpec((CH, D), lambda i: (i, 0)),
            pl.BlockSpec((CH, D), lambda i: (i, 0)),
            _w((D,)), _w((D,)), _w((D, dh)),
            _w((D,)), _w((D,)), _w((D, dh)),
            _w((2 * dh, dh)), _w((dh, 1)),
        ],
        out_specs=pl.BlockSpec((CH, 1, NPAD), lambda i: (i, 0, 0)),
        out_shape=jax.ShapeDtypeStruct((rows, 1, NPAD), jnp.float32),
    )(xr, gf, gs, ln1_g, ln1_b, W1, ln2_g, ln2_b, W2, Wo1, Wo2)

    score2d = score.reshape(rows, NPAD)
    nzp = _noise_sc(rows, n)

    mesh = plsc.VectorSubcoreMesh(core_axis_name="c", subcore_axis_name="s")
    sc_topk = functools.partial(
        pl.kernel,
        out_type=jax.ShapeDtypeStruct((rows, TK * NSC), jnp.float32),
        mesh=mesh,
        compiler_params=pltpu.CompilerParams(needs_layout_passes=False),
        scratch_types=[
            pltpu.VMEM((NPAD,), jnp.float32),
            pltpu.VMEM((NS, NSC), jnp.float32),
            pltpu.VMEM((TK * NSC,), jnp.float32),
        ],
    )(_sc_topk_body)
    cnt = sc_topk(score2d, nzp)

    cnt3 = cnt.reshape(rows, TK, NSC)
    sel = pl.pallas_call(
        _select_body,
        grid=(gsteps,),
        in_specs=[
            pl.BlockSpec((CH, TK, NSC), lambda i: (i, 0, 0)),
            pl.BlockSpec((CH, n, D), lambda i: (i, 0, 0)),
        ],
        out_specs=pl.BlockSpec((CH, TK, D), lambda i: (i, 0, 0)),
        out_shape=jax.ShapeDtypeStruct((rows, TK, D), jnp.float32),
    )(cnt3, xr)

    return sel.reshape(B, MAXF, TK, D)


# speedup vs baseline: 2.6900x; 1.0279x over previous
import functools

import jax
import jax.numpy as jnp
from jax import lax
from jax.experimental import pallas as pl
from jax.experimental.pallas import tpu as pltpu
from jax.experimental.pallas import tpu_sc as plsc

MAXF = 20
TK = 3
NS = 500
SIG = 0.05
NPAD = 256
NSC = 208

_CONSTS = {}


def _noise_sc(rows, n):
    ck = ("sc", rows, n)
    if ck not in _CONSTS:
        with jax.ensure_compile_time_eval():
            nz = jax.random.normal(jax.random.key(1), (rows, NS, n),
                                   dtype=jnp.float32)
            nz = nz * SIG
            pad = jnp.full((rows, NS, NSC - n), -1e30, dtype=jnp.float32)
            _CONSTS[ck] = jnp.concatenate([nz, pad], axis=-1)
    return _CONSTS[ck]


def _ln(v, g, b):
    m = v.mean(axis=-1, keepdims=True)
    var = ((v - m) * (v - m)).mean(axis=-1, keepdims=True)
    return (v - m) / jnp.sqrt(var + 1e-5) * g + b


def _gelu(v):
    return 0.5 * v * (1.0 + jax.lax.erf(v * 0.7071067811865476))


def _score_body(xf_ref, gf_ref, gs_ref, ln1g_ref, ln1b_ref, w1_ref, ln2g_ref,
                ln2b_ref, w2_ref, wo1_ref, wo2_ref, score_ref):
    rows, n, _ = xf_ref.shape
    dh = w1_ref.shape[1]
    xf = xf_ref[...].reshape(rows * n, xf_ref.shape[2])
    xi = _gelu(jnp.dot(_ln(xf, ln1g_ref[...], ln1b_ref[...]), w1_ref[...],
                       preferred_element_type=jnp.float32))

    def head(g_ref):
        gi = _gelu(jnp.dot(_ln(g_ref[...], ln2g_ref[...], ln2b_ref[...]),
                           w2_ref[...], preferred_element_type=jnp.float32))
        gi_b = jnp.broadcast_to(gi[:, None, :], (rows, n, dh)).reshape(rows * n, dh)
        h = jnp.concatenate([xi, gi_b], axis=-1)
        o = _gelu(jnp.dot(h, wo1_ref[...], preferred_element_type=jnp.float32))
        s = jnp.tanh(jnp.dot(o, wo2_ref[...], preferred_element_type=jnp.float32))
        return s.reshape(rows, n)

    sc = head(gf_ref) + head(gs_ref)
    score_ref[...] = jnp.concatenate(
        [sc, jnp.zeros((rows, NPAD - n), dtype=jnp.float32)], axis=-1)[:, None, :]


def _sc_topk_body(score_hbm, noise_hbm, cnt_hbm, score_v, nz_v, cnt_v):
    wid = lax.axis_index("s") * 2 + lax.axis_index("c")
    rows_per_w = 5
    lane = lax.iota(jnp.int32, 16)
    neg = jnp.float32(-3.0e38)
    zeros16 = jnp.zeros((16,), jnp.float32)
    ones16 = jnp.ones((16,), jnp.float32)

    def row_body(r, _carry):
        row = wid * rows_per_w + r
        pltpu.sync_copy(score_hbm.at[row], score_v)
        pltpu.sync_copy(noise_hbm.at[row], nz_v)
        for k in range(3 * NSC // 16):
            cnt_v[pl.ds(16 * k, 16)] = zeros16
        s_vecs = [score_v[pl.ds(16 * j, 16)] for j in range(NSC // 16)]

        def one_sample(s):
            t1 = jnp.full((16,), neg)
            t2 = jnp.full((16,), neg)
            t3 = jnp.full((16,), neg)
            i1 = jnp.zeros((16,), jnp.int32)
            i2 = jnp.zeros((16,), jnp.int32)
            i3 = jnp.zeros((16,), jnp.int32)
            for j in range(NSC // 16):
                v = nz_v[s, pl.ds(16 * j, 16)] + s_vecs[j]
                idx = lane + (16 * j)
                c1 = v > t1
                c2 = v > t2
                c3 = v > t3
                t3n = jnp.where(c2, t2, jnp.where(c3, v, t3))
                i3n = jnp.where(c2, i2, jnp.where(c3, idx, i3))
                t2n = jnp.where(c1, t1, jnp.where(c2, v, t2))
                i2n = jnp.where(c1, i1, jnp.where(c2, idx, i2))
                t1 = jnp.where(c1, v, t1)
                i1 = jnp.where(c1, idx, i1)
                t2, t3, i2, i3 = t2n, t3n, i2n, i3n
            picks = []
            for _k in range(TK):
                m = jnp.max(t1)
                mask = t1 == m
                lane0 = plsc.all_reduce_ffs(mask)
                ikv = jnp.min(jnp.where(mask, i1, 100000))
                picks.append(ikv)
                hit = mask & (lane == lane0)
                t1 = jnp.where(hit, t2, t1)
                i1 = jnp.where(hit, i2, i1)
                t2 = jnp.where(hit, t3, t2)
                i2 = jnp.where(hit, i3, i2)
                t3 = jnp.where(hit, neg, t3)
            a, b, c = picks
            lo = jnp.minimum(jnp.minimum(a, b), c)
            hi = jnp.maximum(jnp.maximum(a, b), c)
            mid = a + b + c - lo - hi
            return jnp.where(lane == 0, lo,
                             jnp.where(lane == 1, mid + NSC, hi + 2 * NSC))

        @plsc.parallel_loop(0, NS // 4, 1, unroll=2)
        def _samp(s):
            for u in range(4):
                iv = one_sample(4 * s + u)
                plsc.addupdate_scatter(cnt_v, [iv], ones16, mask=lane < TK)
        pltpu.sync_copy(cnt_v, cnt_hbm.at[row])
        return _carry

    lax.fori_loop(0, rows_per_w, row_body, 0)


def _select_body(cnt_ref, xr_ref, sel_ref):
    ch, n, _ = xr_ref.shape
    ind = cnt_ref[...] * (1.0 / NS)
    for r in range(ch):
        sel_ref[r] = jnp.dot(ind[r, :, :n], xr_ref[r],
                             preferred_element_type=jnp.float32)


def kernel(x, guidance_frame, guidance_sentence, ln1_g, ln1_b, W1, ln2_g,
           ln2_b, W2, Wo1, Wo2):
    B, L, D = x.shape
    n = L // MAXF
    rows = B * MAXF
    dh = W1.shape[1]

    xr = x.reshape(rows, n, D)
    gf = guidance_frame.reshape(rows, D)
    gs = jnp.broadcast_to(guidance_sentence, (B, MAXF, D)).reshape(rows, D)

    CH = 8
    gsteps = rows // CH

    def _w(shape):
        nd = len(shape)
        return pl.BlockSpec(shape, lambda i, _nd=nd: (0,) * _nd)

    score = pl.pallas_call(
        _score_body,
        grid=(gsteps,),
        in_specs=[
            pl.BlockSpec((CH, n, D), lambda i: (i, 0, 0)),
            pl.BlockSpec((CH, D), lambda i: (i, 0)),
            pl.BlockSpec((CH, D), lambda i: (i, 0)),
            _w((D,)), _w((D,)), _w((D, dh)),
            _w((D,)), _w((D,)), _w((D, dh)),
            _w((2 * dh, dh)), _w((dh, 1)),
        ],
        out_specs=pl.BlockSpec((CH, 1, NPAD), lambda i: (i, 0, 0)),
        out_shape=jax.ShapeDtypeStruct((rows, 1, NPAD), jnp.float32),
    )(xr, gf, gs, ln1_g, ln1_b, W1, ln2_g, ln2_b, W2, Wo1, Wo2)

    score2d = score.reshape(rows, NPAD)
    nzp = _noise_sc(rows, n)

    mesh = plsc.VectorSubcoreMesh(core_axis_name="c", subcore_axis_name="s")
    sc_topk = functools.partial(
        pl.kernel,
        out_type=jax.ShapeDtypeStruct((rows, TK * NSC), jnp.float32),
        mesh=mesh,
        compiler_params=pltpu.CompilerParams(needs_layout_passes=False),
        scratch_types=[
            pltpu.VMEM((NPAD,), jnp.float32),
            pltpu.VMEM((NS, NSC), jnp.float32),
            pltpu.VMEM((TK * NSC,), jnp.float32),
        ],
    )(_sc_topk_body)
    cnt = sc_topk(score2d, nzp)

    cnt3 = cnt.reshape(rows, TK, NSC)
    sel = pl.pallas_call(
        _select_body,
        grid=(gsteps,),
        in_specs=[
            pl.BlockSpec((CH, TK, NSC), lambda i: (i, 0, 0)),
            pl.BlockSpec((CH, n, D), lambda i: (i, 0, 0)),
        ],
        out_specs=pl.BlockSpec((CH, TK, D), lambda i: (i, 0, 0)),
        out_shape=jax.ShapeDtypeStruct((rows, TK, D), jnp.float32),
    )(cnt3, xr)

    return sel.reshape(B, MAXF, TK, D)


# speedup vs baseline: 3.2730x; 1.2168x over previous
import functools

import jax
import jax.numpy as jnp
from jax import lax
from jax.experimental import pallas as pl
from jax.experimental.pallas import tpu as pltpu
from jax.experimental.pallas import tpu_sc as plsc

MAXF = 20
TK = 3
NS = 500
NS_SC = 364
SIG = 0.05
NPAD = 256
NSC = 208

_CONSTS = {}


def _noise_base(rows, n):
    ck = ("base", rows, n)
    if ck not in _CONSTS:
        with jax.ensure_compile_time_eval():
            nz = jax.random.normal(jax.random.key(1), (rows, NS, n),
                                   dtype=jnp.float32)
            _CONSTS[ck] = nz * SIG
    return _CONSTS[ck]


def _noise_sc(rows, n):
    ck = ("sc", rows, n)
    if ck not in _CONSTS:
        with jax.ensure_compile_time_eval():
            nz = _noise_base(rows, n)[:, :NS_SC, :]
            pad = jnp.full((rows, NS_SC, NSC - n), -1e30, dtype=jnp.float32)
            _CONSTS[ck] = jnp.concatenate([nz, pad], axis=-1)
    return _CONSTS[ck]


def _noise_tc(rows, n):
    ck = ("tc", rows, n)
    if ck not in _CONSTS:
        with jax.ensure_compile_time_eval():
            nz = _noise_base(rows, n)[:, NS_SC:, :]
            pad = jnp.full((rows, NS - NS_SC, NPAD - n), -1e30,
                           dtype=jnp.float32)
            _CONSTS[ck] = jnp.concatenate([nz, pad], axis=-1)
    return _CONSTS[ck]


def _ln(v, g, b):
    m = v.mean(axis=-1, keepdims=True)
    var = ((v - m) * (v - m)).mean(axis=-1, keepdims=True)
    return (v - m) / jnp.sqrt(var + 1e-5) * g + b


def _gelu(v):
    return 0.5 * v * (1.0 + jax.lax.erf(v * 0.7071067811865476))


def _score_body(xf_ref, gf_ref, gs_ref, ln1g_ref, ln1b_ref, w1_ref, ln2g_ref,
                ln2b_ref, w2_ref, wo1_ref, wo2_ref, score_ref):
    rows, n, _ = xf_ref.shape
    dh = w1_ref.shape[1]
    xf = xf_ref[...].reshape(rows * n, xf_ref.shape[2])
    xi = _gelu(jnp.dot(_ln(xf, ln1g_ref[...], ln1b_ref[...]), w1_ref[...],
                       preferred_element_type=jnp.float32))

    def head(g_ref):
        gi = _gelu(jnp.dot(_ln(g_ref[...], ln2g_ref[...], ln2b_ref[...]),
                           w2_ref[...], preferred_element_type=jnp.float32))
        gi_b = jnp.broadcast_to(gi[:, None, :], (rows, n, dh)).reshape(rows * n, dh)
        h = jnp.concatenate([xi, gi_b], axis=-1)
        o = _gelu(jnp.dot(h, wo1_ref[...], preferred_element_type=jnp.float32))
        s = jnp.tanh(jnp.dot(o, wo2_ref[...], preferred_element_type=jnp.float32))
        return s.reshape(rows, n)

    sc = head(gf_ref) + head(gs_ref)
    score_ref[...] = jnp.concatenate(
        [sc, jnp.zeros((rows, NPAD - n), dtype=jnp.float32)], axis=-1)[:, None, :]


def _sc_topk_body(score_hbm, noise_hbm, cnt_hbm, score_v, nz_v, cnt_v):
    wid = lax.axis_index("s") * 2 + lax.axis_index("c")
    rows_per_w = 5
    lane = lax.iota(jnp.int32, 16)
    neg = jnp.float32(-3.0e38)
    zeros16 = jnp.zeros((16,), jnp.float32)
    ones16 = jnp.ones((16,), jnp.float32)

    def row_body(r, _carry):
        row = wid * rows_per_w + r
        pltpu.sync_copy(score_hbm.at[row], score_v)
        pltpu.sync_copy(noise_hbm.at[row], nz_v)
        for k in range(3 * NSC // 16):
            cnt_v[pl.ds(16 * k, 16)] = zeros16
        s_vecs = [score_v[pl.ds(16 * j, 16)] for j in range(NSC // 16)]

        def one_sample(s):
            t1 = jnp.full((16,), neg)
            t2 = jnp.full((16,), neg)
            t3 = jnp.full((16,), neg)
            i1 = jnp.zeros((16,), jnp.int32)
            i2 = jnp.zeros((16,), jnp.int32)
            i3 = jnp.zeros((16,), jnp.int32)
            for j in range(NSC // 16):
                v = nz_v[s, pl.ds(16 * j, 16)] + s_vecs[j]
                idx = lane + (16 * j)
                c1 = v > t1
                c2 = v > t2
                c3 = v > t3
                t3n = jnp.where(c2, t2, jnp.where(c3, v, t3))
                i3n = jnp.where(c2, i2, jnp.where(c3, idx, i3))
                t2n = jnp.where(c1, t1, jnp.where(c2, v, t2))
                i2n = jnp.where(c1, i1, jnp.where(c2, idx, i2))
                t1 = jnp.where(c1, v, t1)
                i1 = jnp.where(c1, idx, i1)
                t2, t3, i2, i3 = t2n, t3n, i2n, i3n
            picks = []
            for _k in range(TK):
                m = jnp.max(t1)
                mask = t1 == m
                lane0 = plsc.all_reduce_ffs(mask)
                ikv = jnp.min(jnp.where(mask, i1, 100000))
                picks.append(ikv)
                hit = mask & (lane == lane0)
                t1 = jnp.where(hit, t2, t1)
                i1 = jnp.where(hit, i2, i1)
                t2 = jnp.where(hit, t3, t2)
                i2 = jnp.where(hit, i3, i2)
                t3 = jnp.where(hit, neg, t3)
            a, b, c = picks
            lo = jnp.minimum(jnp.minimum(a, b), c)
            hi = jnp.maximum(jnp.maximum(a, b), c)
            mid = a + b + c - lo - hi
            return jnp.where(lane == 0, lo,
                             jnp.where(lane == 1, mid + NSC, hi + 2 * NSC))

        @plsc.parallel_loop(0, NS_SC // 4, 1, unroll=2)
        def _samp(s):
            for u in range(4):
                iv = one_sample(4 * s + u)
                plsc.addupdate_scatter(cnt_v, [iv], ones16, mask=lane < TK)
        pltpu.sync_copy(cnt_v, cnt_hbm.at[row])
        return _carry

    lax.fori_loop(0, rows_per_w, row_body, 0)


def _tc_cnt_body(sc_ref, nz_ref, cnt_ref):
    ch, s2, npad = nz_ref.shape
    iota = lax.broadcasted_iota(jnp.int32, (s2, npad), 1)
    for r in range(ch):
        p = sc_ref[r] + nz_ref[r]
        picks = []
        for _ in range(TK):
            m = jnp.max(p, axis=1, keepdims=True)
            cand = jnp.where(p == m, iota, npad + 1)
            ik = jnp.min(cand, axis=1)
            picks.append(ik)
            p = jnp.where(iota == ik[:, None], -jnp.inf, p)
        a, b, c = picks
        lo = jnp.minimum(jnp.minimum(a, b), c)
        hi = jnp.maximum(jnp.maximum(a, b), c)
        mid = a + b + c - lo - hi
        cnts = [jnp.sum((ik[:, None] == iota).astype(jnp.float32), axis=0)
                for ik in (lo, mid, hi)]
        cnt_ref[r] = jnp.stack(cnts, axis=0)


def _select_body(cnt_ref, cnt2_ref, xr_ref, sel_ref):
    ch, n, _ = xr_ref.shape
    cs = cnt_ref[...]
    ct = cnt2_ref[...]
    for r in range(ch):
        ind = (cs[r, :, :n] + ct[r, :, :n]) * (1.0 / NS)
        sel_ref[r] = jnp.dot(ind, xr_ref[r],
                             preferred_element_type=jnp.float32)


def kernel(x, guidance_frame, guidance_sentence, ln1_g, ln1_b, W1, ln2_g,
           ln2_b, W2, Wo1, Wo2):
    B, L, D = x.shape
    n = L // MAXF
    rows = B * MAXF
    dh = W1.shape[1]

    xr = x.reshape(rows, n, D)
    gf = guidance_frame.reshape(rows, D)
    gs = jnp.broadcast_to(guidance_sentence, (B, MAXF, D)).reshape(rows, D)

    CH = 8
    gsteps = rows // CH

    def _w(shape):
        nd = len(shape)
        return pl.BlockSpec(shape, lambda i, _nd=nd: (0,) * _nd)

    score = pl.pallas_call(
        _score_body,
        grid=(gsteps,),
        in_specs=[
            pl.BlockSpec((CH, n, D), lambda i: (i, 0, 0)),
            pl.BlockSpec((CH, D), lambda i: (i, 0)),
            pl.BlockSpec((CH, D), lambda i: (i, 0)),
            _w((D,)), _w((D,)), _w((D, dh)),
            _w((D,)), _w((D,)), _w((D, dh)),
            _w((2 * dh, dh)), _w((dh, 1)),
        ],
        out_specs=pl.BlockSpec((CH, 1, NPAD), lambda i: (i, 0, 0)),
        out_shape=jax.ShapeDtypeStruct((rows, 1, NPAD), jnp.float32),
    )(xr, gf, gs, ln1_g, ln1_b, W1, ln2_g, ln2_b, W2, Wo1, Wo2)

    score2d = score.reshape(rows, NPAD)
    nzp = _noise_sc(rows, n)
    nzt = _noise_tc(rows, n)

    mesh = plsc.VectorSubcoreMesh(core_axis_name="c", subcore_axis_name="s")
    sc_topk = functools.partial(
        pl.kernel,
        out_type=jax.ShapeDtypeStruct((rows, TK * NSC), jnp.float32),
        mesh=mesh,
        compiler_params=pltpu.CompilerParams(needs_layout_passes=False),
        scratch_types=[
            pltpu.VMEM((NPAD,), jnp.float32),
            pltpu.VMEM((NS_SC, NSC), jnp.float32),
            pltpu.VMEM((TK * NSC,), jnp.float32),
        ],
    )(_sc_topk_body)
    cnt = sc_topk(score2d, nzp)

    cnt_tc = pl.pallas_call(
        _tc_cnt_body,
        grid=(gsteps,),
        in_specs=[
            pl.BlockSpec((CH, 1, NPAD), lambda i: (i, 0, 0)),
            pl.BlockSpec((CH, NS - NS_SC, NPAD), lambda i: (i, 0, 0)),
        ],
        out_specs=pl.BlockSpec((CH, TK, NPAD), lambda i: (i, 0, 0)),
        out_shape=jax.ShapeDtypeStruct((rows, TK, NPAD), jnp.float32),
    )(score, nzt)

    cnt3 = cnt.reshape(rows, TK, NSC)
    sel = pl.pallas_call(
        _select_body,
        grid=(gsteps,),
        in_specs=[
            pl.BlockSpec((CH, TK, NSC), lambda i: (i, 0, 0)),
            pl.BlockSpec((CH, TK, NPAD), lambda i: (i, 0, 0)),
            pl.BlockSpec((CH, n, D), lambda i: (i, 0, 0)),
        ],
        out_specs=pl.BlockSpec((CH, TK, D), lambda i: (i, 0, 0)),
        out_shape=jax.ShapeDtypeStruct((rows, TK, D), jnp.float32),
    )(cnt3, cnt_tc, xr)

    return sel.reshape(B, MAXF, TK, D)


# speedup vs baseline: 3.8160x; 1.1659x over previous
import functools

import jax
import jax.numpy as jnp
from jax import lax
from jax.experimental import pallas as pl
from jax.experimental.pallas import tpu as pltpu
from jax.experimental.pallas import tpu_sc as plsc

MAXF = 20
TK = 3
NS = 500
NS_SC = 280
SIG = 0.05
NPAD = 256
NSC = 208

_CONSTS = {}


def _noise_base(rows, n):
    ck = ("base", rows, n)
    if ck not in _CONSTS:
        with jax.ensure_compile_time_eval():
            nz = jax.random.normal(jax.random.key(1), (rows, NS, n),
                                   dtype=jnp.float32)
            _CONSTS[ck] = nz * SIG
    return _CONSTS[ck]


def _noise_sc(rows, n):
    ck = ("sc", rows, n)
    if ck not in _CONSTS:
        with jax.ensure_compile_time_eval():
            nz = _noise_base(rows, n)[:, :NS_SC, :]
            pad = jnp.full((rows, NS_SC, NSC - n), -1e30, dtype=jnp.float32)
            _CONSTS[ck] = jnp.concatenate([nz, pad], axis=-1)
    return _CONSTS[ck]


def _noise_tc(rows, n):
    ck = ("tc", rows, n)
    if ck not in _CONSTS:
        with jax.ensure_compile_time_eval():
            nz = _noise_base(rows, n)[:, NS_SC:, :]
            pad = jnp.full((rows, NS - NS_SC, NPAD - n), -1e30,
                           dtype=jnp.float32)
            _CONSTS[ck] = jnp.concatenate([nz, pad], axis=-1)
    return _CONSTS[ck]


def _ln(v, g, b):
    m = v.mean(axis=-1, keepdims=True)
    var = ((v - m) * (v - m)).mean(axis=-1, keepdims=True)
    return (v - m) / jnp.sqrt(var + 1e-5) * g + b


def _gelu(v):
    return 0.5 * v * (1.0 + jax.lax.erf(v * 0.7071067811865476))


def _score_body(xf_ref, gf_ref, gs_ref, ln1g_ref, ln1b_ref, w1_ref, ln2g_ref,
                ln2b_ref, w2_ref, wo1_ref, wo2_ref, score_ref):
    rows, n, _ = xf_ref.shape
    dh = w1_ref.shape[1]
    xf = xf_ref[...].reshape(rows * n, xf_ref.shape[2])
    xi = _gelu(jnp.dot(_ln(xf, ln1g_ref[...], ln1b_ref[...]), w1_ref[...],
                       preferred_element_type=jnp.float32))

    def head(g_ref):
        gi = _gelu(jnp.dot(_ln(g_ref[...], ln2g_ref[...], ln2b_ref[...]),
                           w2_ref[...], preferred_element_type=jnp.float32))
        gi_b = jnp.broadcast_to(gi[:, None, :], (rows, n, dh)).reshape(rows * n, dh)
        h = jnp.concatenate([xi, gi_b], axis=-1)
        o = _gelu(jnp.dot(h, wo1_ref[...], preferred_element_type=jnp.float32))
        s = jnp.tanh(jnp.dot(o, wo2_ref[...], preferred_element_type=jnp.float32))
        return s.reshape(rows, n)

    sc = head(gf_ref) + head(gs_ref)
    score_ref[...] = jnp.concatenate(
        [sc, jnp.zeros((rows, NPAD - n), dtype=jnp.float32)], axis=-1)[:, None, :]


def _sc_topk_body(score_hbm, noise_hbm, cnt_hbm, score_v, nz_v, cnt_v):
    wid = lax.axis_index("s") * 2 + lax.axis_index("c")
    rows_per_w = 5
    lane = lax.iota(jnp.int32, 16)
    neg = jnp.float32(-3.0e38)
    zeros16 = jnp.zeros((16,), jnp.float32)
    ones16 = jnp.ones((16,), jnp.float32)

    def row_body(r, _carry):
        row = wid * rows_per_w + r
        pltpu.sync_copy(score_hbm.at[row], score_v)
        pltpu.sync_copy(noise_hbm.at[row], nz_v)
        for k in range(3 * NSC // 16):
            cnt_v[pl.ds(16 * k, 16)] = zeros16
        s_vecs = [score_v[pl.ds(16 * j, 16)] for j in range(NSC // 16)]

        def one_sample(s):
            t1 = jnp.full((16,), neg)
            t2 = jnp.full((16,), neg)
            t3 = jnp.full((16,), neg)
            i1 = jnp.zeros((16,), jnp.int32)
            i2 = jnp.zeros((16,), jnp.int32)
            i3 = jnp.zeros((16,), jnp.int32)
            for j in range(NSC // 16):
                v = nz_v[s, pl.ds(16 * j, 16)] + s_vecs[j]
                idx = lane + (16 * j)
                c1 = v > t1
                c2 = v > t2
                c3 = v > t3
                t3n = jnp.where(c2, t2, jnp.where(c3, v, t3))
                i3n = jnp.where(c2, i2, jnp.where(c3, idx, i3))
                t2n = jnp.where(c1, t1, jnp.where(c2, v, t2))
                i2n = jnp.where(c1, i1, jnp.where(c2, idx, i2))
                t1 = jnp.where(c1, v, t1)
                i1 = jnp.where(c1, idx, i1)
                t2, t3, i2, i3 = t2n, t3n, i2n, i3n
            picks = []
            for _k in range(TK):
                m = jnp.max(t1)
                mask = t1 == m
                lane0 = plsc.all_reduce_ffs(mask)
                ikv = jnp.min(jnp.where(mask, i1, 100000))
                picks.append(ikv)
                hit = mask & (lane == lane0)
                t1 = jnp.where(hit, t2, t1)
                i1 = jnp.where(hit, i2, i1)
                t2 = jnp.where(hit, t3, t2)
                i2 = jnp.where(hit, i3, i2)
                t3 = jnp.where(hit, neg, t3)
            a, b, c = picks
            lo = jnp.minimum(jnp.minimum(a, b), c)
            hi = jnp.maximum(jnp.maximum(a, b), c)
            mid = a + b + c - lo - hi
            return jnp.where(lane == 0, lo,
                             jnp.where(lane == 1, mid + NSC, hi + 2 * NSC))

        @plsc.parallel_loop(0, NS_SC // 4, 1, unroll=2)
        def _samp(s):
            for u in range(4):
                iv = one_sample(4 * s + u)
                plsc.addupdate_scatter(cnt_v, [iv], ones16, mask=lane < TK)
        pltpu.sync_copy(cnt_v, cnt_hbm.at[row])
        return _carry

    lax.fori_loop(0, rows_per_w, row_body, 0)


def _tc_cnt_body(sc_ref, nz_ref, cnt_ref):
    ch, s2, npad = nz_ref.shape
    iota = lax.broadcasted_iota(jnp.int32, (s2, npad), 1)
    for r in range(ch):
        p = sc_ref[r] + nz_ref[r]
        picks = []
        for _ in range(TK):
            m = jnp.max(p, axis=1, keepdims=True)
            cand = jnp.where(p == m, iota, npad + 1)
            ik = jnp.min(cand, axis=1)
            picks.append(ik)
            p = jnp.where(iota == ik[:, None], -jnp.inf, p)
        a, b, c = picks
        lo = jnp.minimum(jnp.minimum(a, b), c)
        hi = jnp.maximum(jnp.maximum(a, b), c)
        mid = a + b + c - lo - hi
        cnts = [jnp.sum((ik[:, None] == iota).astype(jnp.float32), axis=0)
                for ik in (lo, mid, hi)]
        cnt_ref[r] = jnp.stack(cnts, axis=0)


def _select_body(cnt_ref, cnt2_ref, xr_ref, sel_ref):
    ch, n, _ = xr_ref.shape
    cs = cnt_ref[...]
    ct = cnt2_ref[...]
    for r in range(ch):
        ind = (cs[r, :, :n] + ct[r, :, :n]) * (1.0 / NS)
        sel_ref[r] = jnp.dot(ind, xr_ref[r],
                             preferred_element_type=jnp.float32)


def kernel(x, guidance_frame, guidance_sentence, ln1_g, ln1_b, W1, ln2_g,
           ln2_b, W2, Wo1, Wo2):
    B, L, D = x.shape
    n = L // MAXF
    rows = B * MAXF
    dh = W1.shape[1]

    xr = x.reshape(rows, n, D)
    gf = guidance_frame.reshape(rows, D)
    gs = jnp.broadcast_to(guidance_sentence, (B, MAXF, D)).reshape(rows, D)

    CH = 8
    gsteps = rows // CH

    def _w(shape):
        nd = len(shape)
        return pl.BlockSpec(shape, lambda i, _nd=nd: (0,) * _nd)

    score = pl.pallas_call(
        _score_body,
        grid=(gsteps,),
        in_specs=[
            pl.BlockSpec((CH, n, D), lambda i: (i, 0, 0)),
            pl.BlockSpec((CH, D), lambda i: (i, 0)),
            pl.BlockSpec((CH, D), lambda i: (i, 0)),
            _w((D,)), _w((D,)), _w((D, dh)),
            _w((D,)), _w((D,)), _w((D, dh)),
            _w((2 * dh, dh)), _w((dh, 1)),
        ],
        out_specs=pl.BlockSpec((CH, 1, NPAD), lambda i: (i, 0, 0)),
        out_shape=jax.ShapeDtypeStruct((rows, 1, NPAD), jnp.float32),
    )(xr, gf, gs, ln1_g, ln1_b, W1, ln2_g, ln2_b, W2, Wo1, Wo2)

    score2d = score.reshape(rows, NPAD)
    nzp = _noise_sc(rows, n)
    nzt = _noise_tc(rows, n)

    mesh = plsc.VectorSubcoreMesh(core_axis_name="c", subcore_axis_name="s")
    sc_topk = functools.partial(
        pl.kernel,
        out_type=jax.ShapeDtypeStruct((rows, TK * NSC), jnp.float32),
        mesh=mesh,
        compiler_params=pltpu.CompilerParams(needs_layout_passes=False),
        scratch_types=[
            pltpu.VMEM((NPAD,), jnp.float32),
            pltpu.VMEM((NS_SC, NSC), jnp.float32),
            pltpu.VMEM((TK * NSC,), jnp.float32),
        ],
    )(_sc_topk_body)
    cnt = sc_topk(score2d, nzp)

    cnt_tc = pl.pallas_call(
        _tc_cnt_body,
        grid=(gsteps,),
        in_specs=[
            pl.BlockSpec((CH, 1, NPAD), lambda i: (i, 0, 0)),
            pl.BlockSpec((CH, NS - NS_SC, NPAD), lambda i: (i, 0, 0)),
        ],
        out_specs=pl.BlockSpec((CH, TK, NPAD), lambda i: (i, 0, 0)),
        out_shape=jax.ShapeDtypeStruct((rows, TK, NPAD), jnp.float32),
    )(score, nzt)

    cnt3 = cnt.reshape(rows, TK, NSC)
    sel = pl.pallas_call(
        _select_body,
        grid=(gsteps,),
        in_specs=[
            pl.BlockSpec((CH, TK, NSC), lambda i: (i, 0, 0)),
            pl.BlockSpec((CH, TK, NPAD), lambda i: (i, 0, 0)),
            pl.BlockSpec((CH, n, D), lambda i: (i, 0, 0)),
        ],
        out_specs=pl.BlockSpec((CH, TK, D), lambda i: (i, 0, 0)),
        out_shape=jax.ShapeDtypeStruct((rows, TK, D), jnp.float32),
    )(cnt3, cnt_tc, xr)

    return sel.reshape(B, MAXF, TK, D)
